# baseline (device time: 169289 ns/iter reference)
import functools

import jax
import jax.numpy as jnp
from jax import lax
from jax.experimental import pallas as pl
from jax.experimental.pallas import tpu as pltpu

N_DEV = 32
M = 2048
N = 2048
HALF = M // 2
QTR = HALF // 4
ZC = QTR // 4
XPIECE = HALF // 2

F32 = jnp.float32
BF16 = jnp.bfloat16

XRS = (0, 1, 2, 3)
YRS = (4, 5, 6)
ZRS = (7, 8, 9)
ZAG = (10, 11, 12)
YAG = (13, 14, 15)
XAG = (16, 17, 18, 19)


def kernel(A, B):
    def body(
        a_ref,
        b_ref,
        out_ref,
        xsend_ref,
        xrecv_ref,
        ysend_ref,
        yrecv_ref,
        qbuf_ref,
        zsend_ref,
        zrecv_ref,
        gath_ref,
        xagrecv_ref,
        send_sems,
        recv_sems,
    ):
        me = lax.axis_index("i")
        z = me // 8
        q = lax.rem(me, 8)
        y = q // 2
        r4 = lax.rem(q, 4)
        x = jnp.where((r4 == 1) | (r4 == 2), 1, 0)

        def q_of(x_, y_):
            return 2 * y_ + jnp.where(lax.rem(y_, 2) == 0, x_, 1 - x_)

        x_partner = z * 8 + (q + 1 - 2 * lax.rem(q, 2))
        y_next = z * 8 + q_of(x, lax.rem(y + 1, 4))
        y_prev = z * 8 + q_of(x, lax.rem(y + 3, 4))
        z_next = lax.rem(z + 1, 4) * 8 + q
        z_prev = lax.rem(z + 3, 4) * 8 + q

        myrow0 = x * HALF
        prow0 = (1 - x) * HALF

        barrier_sem = pltpu.get_barrier_semaphore()
        for nbr in (x_partner, y_next, y_prev, z_next, z_prev):
            pl.semaphore_signal(
                barrier_sem, inc=1,
                device_id=(nbr,), device_id_type=pl.DeviceIdType.MESH,
            )
        pl.semaphore_wait(barrier_sem, 5)

        def rdma(src, dst, idx, target):
            return pltpu.make_async_remote_copy(
                src_ref=src,
                dst_ref=dst,
                send_sem=send_sems.at[idx],
                recv_sem=recv_sems.at[idx],
                device_id=(target,),
                device_id_type=pl.DeviceIdType.MESH,
            )

        def part_rows(row0, nrows):
            return jnp.dot(
                a_ref[pl.ds(row0, nrows), :],
                b_ref[...],
                preferred_element_type=F32,
            )

        xrdmas = []
        for j in range(4):
            cj = lax.rem(y + (4 - j), 4)
            xsend_ref[j] = part_rows(prow0 + cj * QTR, QTR).astype(BF16)
            rd = rdma(
                xsend_ref.at[j], xrecv_ref.at[j], XRS[j], x_partner
            )
            rd.start()
            xrdmas.append(rd)

        part0 = part_rows(myrow0 + y * QTR, QTR)

        xrdmas[0].wait()
        ysend_ref[...] = (
            xrecv_ref[0].astype(F32) + part0
        ).astype(BF16)
        for s in range(3):
            rd = rdma(ysend_ref, yrecv_ref.at[s], YRS[s], y_next)
            rd.start()
            c = lax.rem(y + (3 - s), 4)
            partv = part_rows(myrow0 + c * QTR, QTR)
            xrdmas[s + 1].wait()
            rd.wait()
            acc = (
                yrecv_ref[s].astype(F32)
                + xrecv_ref[s + 1].astype(F32)
                + partv
            )
            if s < 2:
                ysend_ref[...] = acc.astype(BF16)
            else:
                qbuf_ref[...] = acc

        yc_own = lax.rem(y + 1, 4)
        qrow0 = myrow0 + yc_own * QTR

        zsend_ref[...] = qbuf_ref[pl.ds(z * ZC, ZC), :].astype(BF16)
        for s in range(3):
            rd = rdma(zsend_ref, zrecv_ref.at[s], ZRS[s], z_next)
            rd.start()
            c = lax.rem(z + (3 - s), 4)
            rd.wait()
            acc = zrecv_ref[s].astype(F32) + qbuf_ref[pl.ds(c * ZC, ZC), :]
            if s < 2:
                zsend_ref[...] = acc.astype(BF16)
            else:
                fin = jnp.maximum(acc, 0.0)
                zc_own = lax.rem(z + 1, 4)
                loc0 = yc_own * QTR + zc_own * ZC
                out_ref[pl.ds(myrow0 + loc0, ZC), :] = fin
                gath_ref[pl.ds(loc0, ZC), :] = fin.astype(BF16)

        for t in range(3):
            zc_s = lax.rem(z + (5 - t), 4)
            src0 = yc_own * QTR + zc_s * ZC
            rd = rdma(
                gath_ref.at[pl.ds(src0, ZC), :],
                gath_ref.at[pl.ds(src0, ZC), :],
                ZAG[t],
                z_next,
            )
            rd.start()
            rd.wait()
            zc_r = lax.rem(z + (4 - t), 4)
            loc = yc_own * QTR + zc_r * ZC
            out_ref[pl.ds(myrow0 + loc, ZC), :] = (
                gath_ref[pl.ds(loc, ZC), :].astype(F32)
            )

        def xag_start(piece_idx, yc):
            rd = rdma(
                gath_ref.at[pl.ds(yc * QTR, QTR), :],
                xagrecv_ref.at[pl.ds(yc * QTR, QTR), :],
                XAG[piece_idx],
                x_partner,
            )
            rd.start()
            return rd, yc

        xag_pending = [xag_start(0, yc_own)]
        for t in range(3):
            yc_s = lax.rem(y + (5 - t), 4)
            rd = rdma(
                gath_ref.at[pl.ds(yc_s * QTR, QTR), :],
                gath_ref.at[pl.ds(yc_s * QTR, QTR), :],
                YAG[t],
                y_next,
            )
            rd.start()
            rd.wait()
            yc_r = lax.rem(y + (4 - t), 4)
            out_ref[pl.ds(myrow0 + yc_r * QTR, QTR), :] = (
                gath_ref[pl.ds(yc_r * QTR, QTR), :].astype(F32)
            )
            xag_pending.append(xag_start(t + 1, yc_r))
            rd_old, yc_old = xag_pending.pop(0)
            rd_old.wait()
            out_ref[pl.ds(prow0 + yc_old * QTR, QTR), :] = (
                xagrecv_ref[pl.ds(yc_old * QTR, QTR), :].astype(F32)
            )
        rd_old, yc_old = xag_pending.pop(0)
        rd_old.wait()
        out_ref[pl.ds(prow0 + yc_old * QTR, QTR), :] = (
            xagrecv_ref[pl.ds(yc_old * QTR, QTR), :].astype(F32)
        )

        @functools.partial(
            pl.run_scoped, second_barrier=pltpu.SemaphoreType.REGULAR
        )
        def _(second_barrier):
            for nbr in (x_partner, y_next, y_prev, z_next, z_prev):
                pl.semaphore_signal(
                    second_barrier, inc=1,
                    device_id=(nbr,), device_id_type=pl.DeviceIdType.MESH,
                )
            pl.semaphore_wait(second_barrier, 5)

    return pl.pallas_call(
        body,
        out_shape=jax.ShapeDtypeStruct((M, N), F32),
        in_specs=[
            pl.BlockSpec(memory_space=pltpu.VMEM),
            pl.BlockSpec(memory_space=pltpu.VMEM),
        ],
        out_specs=pl.BlockSpec(memory_space=pltpu.VMEM),
        scratch_shapes=[
            pltpu.VMEM((4, QTR, N), BF16),
            pltpu.VMEM((4, QTR, N), BF16),
            pltpu.VMEM((QTR, N), BF16),
            pltpu.VMEM((3, QTR, N), BF16),
            pltpu.VMEM((QTR, N), F32),
            pltpu.VMEM((ZC, N), BF16),
            pltpu.VMEM((3, ZC, N), BF16),
            pltpu.VMEM((HALF, N), BF16),
            pltpu.VMEM((HALF, N), BF16),
            pltpu.SemaphoreType.DMA((20,)),
            pltpu.SemaphoreType.DMA((20,)),
        ],
        compiler_params=pltpu.CompilerParams(
            collective_id=0, vmem_limit_bytes=100 * 1024 * 1024
        ),
    )(A, B)


# device time: 162934 ns/iter; 1.0390x vs baseline; 1.0390x over previous
import functools

import jax
import jax.numpy as jnp
from jax import lax
from jax.experimental import pallas as pl
from jax.experimental.pallas import tpu as pltpu

M = 2048
N = 2048
HN = N // 2
HALF = M // 2
QTR = HALF // 4
ZC = QTR // 4

F32 = jnp.float32
BF16 = jnp.bfloat16

XRS = (0, 1, 2, 3)
YRS = (4, 5, 6)
ZRS = (7, 8, 9)
ZAG = (10, 11, 12)
YAG = (13, 14, 15)
XAG = (16, 17, 18, 19)


def kernel(A, B):
    def body(
        a_ref,
        b_ref,
        out_ref,
        xsend_ref,
        xrecv_ref,
        ysend_ref,
        yrecv_ref,
        qbuf_ref,
        zsend_ref,
        zrecv_ref,
        zagbuf_ref,
        yagbuf_ref,
        xagrecv_ref,
        send_sems,
        recv_sems,
    ):
        me = lax.axis_index("i")
        z = me // 8
        q = lax.rem(me, 8)
        y = q // 2
        r4 = lax.rem(q, 4)
        x = jnp.where((r4 == 1) | (r4 == 2), 1, 0)

        def q_of(x_, y_):
            return 2 * y_ + jnp.where(lax.rem(y_, 2) == 0, x_, 1 - x_)

        x_partner = z * 8 + (q + 1 - 2 * lax.rem(q, 2))
        y_next = z * 8 + q_of(x, lax.rem(y + 1, 4))
        y_prev = z * 8 + q_of(x, lax.rem(y + 3, 4))
        z_next = lax.rem(z + 1, 4) * 8 + q
        z_prev = lax.rem(z + 3, 4) * 8 + q

        myrow0 = x * HALF
        prow0 = (1 - x) * HALF
        yc_own = lax.rem(y + 1, 4)
        zc_own = lax.rem(z + 1, 4)

        barrier_sem = pltpu.get_barrier_semaphore()
        for nbr in (x_partner, y_next, y_prev, z_next, z_prev):
            pl.semaphore_signal(
                barrier_sem, inc=1,
                device_id=(nbr,), device_id_type=pl.DeviceIdType.MESH,
            )
        pl.semaphore_wait(barrier_sem, 5)

        def rdma(src, dst, p, idx, target):
            return pltpu.make_async_remote_copy(
                src_ref=src,
                dst_ref=dst,
                send_sem=send_sems.at[p * 20 + idx],
                recv_sem=recv_sems.at[p * 20 + idx],
                device_id=(target,),
                device_id_type=pl.DeviceIdType.MESH,
            )

        def part_rows(p, row0, nrows):
            return jnp.dot(
                a_ref[pl.ds(row0, nrows), :],
                b_ref[:, pl.ds(p * HN, HN)],
                preferred_element_type=F32,
            )

        def xrs_start(p):
            rds = []
            for j in range(4):
                cj = lax.rem(y + (4 - j), 4)
                xsend_ref[p, j] = (
                    part_rows(p, prow0 + cj * QTR, QTR).astype(BF16)
                )
                rd = rdma(
                    xsend_ref.at[p, j], xrecv_ref.at[p, j],
                    p, XRS[j], x_partner,
                )
                rd.start()
                rds.append(rd)
            return rds

        def y_init(p, xr):
            xr[0].wait()
            part0 = part_rows(p, myrow0 + y * QTR, QTR)
            ysend_ref[p] = (xrecv_ref[p, 0].astype(F32) + part0).astype(BF16)

        def y_step_start(p, s):
            rd = rdma(
                ysend_ref.at[p], yrecv_ref.at[p, s], p, YRS[s], y_next
            )
            rd.start()
            return rd

        def y_step_finish(p, s, rd, xr):
            c = lax.rem(y + (3 - s), 4)
            partv = part_rows(p, myrow0 + c * QTR, QTR)
            xr[s + 1].wait()
            rd.wait()
            acc = (
                yrecv_ref[p, s].astype(F32)
                + xrecv_ref[p, s + 1].astype(F32)
                + partv
            )
            if s < 2:
                ysend_ref[p] = acc.astype(BF16)
            else:
                qbuf_ref[p] = acc

        def z_init(p):
            zsend_ref[p] = qbuf_ref[p, pl.ds(z * ZC, ZC), :].astype(BF16)

        def z_step_start(p, s):
            rd = rdma(
                zsend_ref.at[p], zrecv_ref.at[p, s], p, ZRS[s], z_next
            )
            rd.start()
            return rd

        def z_step_finish(p, s, rd):
            rd.wait()
            c = lax.rem(z + (3 - s), 4)
            acc = (
                zrecv_ref[p, s].astype(F32)
                + qbuf_ref[p, pl.ds(c * ZC, ZC), :]
            )
            if s < 2:
                zsend_ref[p] = acc.astype(BF16)
            else:
                fin = jnp.maximum(acc, 0.0)
                g0 = myrow0 + yc_own * QTR + zc_own * ZC
                out_ref[pl.ds(g0, ZC), pl.ds(p * HN, HN)] = fin
                zagbuf_ref[p, 0] = fin.astype(BF16)

        def zag_start(p, t):
            rd = rdma(
                zagbuf_ref.at[p, t], zagbuf_ref.at[p, t + 1],
                p, ZAG[t], z_next,
            )
            rd.start()
            return rd

        def zag_finish(p, t, rd):
            rd.wait()
            zc_r = lax.rem(z + (4 - t), 4)
            out_ref[
                pl.ds(myrow0 + yc_own * QTR + zc_r * ZC, ZC),
                pl.ds(p * HN, HN),
            ] = zagbuf_ref[p, t + 1].astype(F32)

        def assemble_quarter(p):
            for k in range(4):
                zc = lax.rem(z + (1 - k) + 4, 4)
                yagbuf_ref[p, 0, pl.ds(zc * ZC, ZC), :] = zagbuf_ref[p, k]

        def yag_start(p, t):
            rd = rdma(
                yagbuf_ref.at[p, t], yagbuf_ref.at[p, t + 1],
                p, YAG[t], y_next,
            )
            rd.start()
            return rd

        def yag_finish(p, t, rd):
            rd.wait()
            yc_r = lax.rem(y + (4 - t), 4)
            out_ref[
                pl.ds(myrow0 + yc_r * QTR, QTR), pl.ds(p * HN, HN)
            ] = yagbuf_ref[p, t + 1].astype(F32)

        def xag_start(p, t):
            rd = rdma(
                yagbuf_ref.at[p, t], xagrecv_ref.at[p, t],
                p, XAG[t], x_partner,
            )
            rd.start()
            return rd

        def xag_finish(p, t, rd):
            rd.wait()
            if t == 0:
                pc = yc_own
            else:
                pc = lax.rem(y + (5 - t), 4)
            out_ref[
                pl.ds(prow0 + pc * QTR, QTR), pl.ds(p * HN, HN)
            ] = xagrecv_ref[p, t].astype(F32)

        xr0 = xrs_start(0)
        xr1 = xrs_start(1)

        y_init(0, xr0)
        for s in range(3):
            rd = y_step_start(0, s)
            y_step_finish(0, s, rd, xr0)

        y_init(1, xr1)
        z_init(0)
        for k in range(3):
            yrd = y_step_start(1, k)
            zrd = z_step_start(0, k)
            z_step_finish(0, k, zrd)
            y_step_finish(1, k, yrd, xr1)

        z_init(1)
        for k in range(3):
            zagrd = zag_start(0, k)
            zrd = z_step_start(1, k)
            zag_finish(0, k, zagrd)
            z_step_finish(1, k, zrd)
        assemble_quarter(0)

        xag_pending0 = xag_start(0, 0)
        for k in range(3):
            yagrd = yag_start(0, k)
            zagrd = zag_start(1, k)
            yag_finish(0, k, yagrd)
            nxt = xag_start(0, k + 1)
            xag_finish(0, k, xag_pending0)
            xag_pending0 = nxt
            zag_finish(1, k, zagrd)
        assemble_quarter(1)
        xag_finish(0, 3, xag_pending0)

        xag_pending1 = xag_start(1, 0)
        for k in range(3):
            yagrd = yag_start(1, k)
            yag_finish(1, k, yagrd)
            nxt = xag_start(1, k + 1)
            xag_finish(1, k, xag_pending1)
            xag_pending1 = nxt
        xag_finish(1, 3, xag_pending1)

        @functools.partial(
            pl.run_scoped, second_barrier=pltpu.SemaphoreType.REGULAR
        )
        def _(second_barrier):
            for nbr in (x_partner, y_next, y_prev, z_next, z_prev):
                pl.semaphore_signal(
                    second_barrier, inc=1,
                    device_id=(nbr,), device_id_type=pl.DeviceIdType.MESH,
                )
            pl.semaphore_wait(second_barrier, 5)

    return pl.pallas_call(
        body,
        out_shape=jax.ShapeDtypeStruct((M, N), F32),
        in_specs=[
            pl.BlockSpec(memory_space=pltpu.VMEM),
            pl.BlockSpec(memory_space=pltpu.VMEM),
        ],
        out_specs=pl.BlockSpec(memory_space=pltpu.VMEM),
        scratch_shapes=[
            pltpu.VMEM((2, 4, QTR, HN), BF16),
            pltpu.VMEM((2, 4, QTR, HN), BF16),
            pltpu.VMEM((2, QTR, HN), BF16),
            pltpu.VMEM((2, 3, QTR, HN), BF16),
            pltpu.VMEM((2, QTR, HN), F32),
            pltpu.VMEM((2, ZC, HN), BF16),
            pltpu.VMEM((2, 3, ZC, HN), BF16),
            pltpu.VMEM((2, 4, ZC, HN), BF16),
            pltpu.VMEM((2, 4, QTR, HN), BF16),
            pltpu.VMEM((2, 4, QTR, HN), BF16),
            pltpu.SemaphoreType.DMA((40,)),
            pltpu.SemaphoreType.DMA((40,)),
        ],
        compiler_params=pltpu.CompilerParams(
            collective_id=0, vmem_limit_bytes=100 * 1024 * 1024
        ),
    )(A, B)


# device time: 160001 ns/iter; 1.0580x vs baseline; 1.0183x over previous
import functools

import jax
import jax.numpy as jnp
from jax import lax
from jax.experimental import pallas as pl
from jax.experimental.pallas import tpu as pltpu

M = 2048
N = 2048
HN = N // 2
HALF = M // 2
QTR = HALF // 4
ZC = QTR // 4

F32 = jnp.float32
BF16 = jnp.bfloat16

XRS = (0, 1, 2, 3)
YRS = (4, 5, 6)
ZRS = (7, 8, 9)
ZAG = (10, 11, 12)
YAG = (13, 14, 15)
XAG = (16, 17, 18, 19)


def kernel(A, B):
    def body(
        a_ref,
        b_ref,
        out_ref,
        xsend_ref,
        xrecv_ref,
        ysend_ref,
        yrecv_ref,
        qbuf_ref,
        zsend_ref,
        zrecv_ref,
        zagbuf_ref,
        yagbuf_ref,
        xagrecv_ref,
        send_sems,
        recv_sems,
    ):
        me = lax.axis_index("i")
        z = me // 8
        q = lax.rem(me, 8)
        y = q // 2
        r4 = lax.rem(q, 4)
        x = jnp.where((r4 == 1) | (r4 == 2), 1, 0)

        def q_of(x_, y_):
            return 2 * y_ + jnp.where(lax.rem(y_, 2) == 0, x_, 1 - x_)

        x_partner = z * 8 + (q + 1 - 2 * lax.rem(q, 2))
        y_next = z * 8 + q_of(x, lax.rem(y + 1, 4))
        y_prev = z * 8 + q_of(x, lax.rem(y + 3, 4))
        z_next = lax.rem(z + 1, 4) * 8 + q
        z_prev = lax.rem(z + 3, 4) * 8 + q

        myrow0 = x * HALF
        prow0 = (1 - x) * HALF
        yc_own = lax.rem(y + 1, 4)
        zc_own = lax.rem(z + 1, 4)

        barrier_sem = pltpu.get_barrier_semaphore()
        for nbr in (x_partner, y_next, y_prev, z_next, z_prev):
            pl.semaphore_signal(
                barrier_sem, inc=1,
                device_id=(nbr,), device_id_type=pl.DeviceIdType.MESH,
            )
        pl.semaphore_wait(barrier_sem, 5)

        def rdma(src, dst, p, idx, target):
            return pltpu.make_async_remote_copy(
                src_ref=src,
                dst_ref=dst,
                send_sem=send_sems.at[p * 20 + idx],
                recv_sem=recv_sems.at[p * 20 + idx],
                device_id=(target,),
                device_id_type=pl.DeviceIdType.MESH,
            )

        def part_rows(p, row0, nrows):
            return jnp.dot(
                a_ref[pl.ds(row0, nrows), :],
                b_ref[:, pl.ds(p * HN, HN)],
                preferred_element_type=F32,
            )

        def xrs_start(p):
            rds = []
            for j in range(4):
                cj = lax.rem(y + (4 - j), 4)
                xsend_ref[p, j] = (
                    part_rows(p, prow0 + cj * QTR, QTR).astype(BF16)
                )
                rd = rdma(
                    xsend_ref.at[p, j], xrecv_ref.at[p, j],
                    p, XRS[j], x_partner,
                )
                rd.start()
                rds.append(rd)
            return rds

        def y_init(p, xr):
            xr[0].wait()
            part0 = part_rows(p, myrow0 + y * QTR, QTR)
            ysend_ref[p] = (xrecv_ref[p, 0].astype(F32) + part0).astype(BF16)

        def y_step_start(p, s):
            rd = rdma(
                ysend_ref.at[p], yrecv_ref.at[p, s], p, YRS[s], y_next
            )
            rd.start()
            return rd

        def y_step_finish(p, s, rd, xr):
            c = lax.rem(y + (3 - s), 4)
            partv = part_rows(p, myrow0 + c * QTR, QTR)
            xr[s + 1].wait()
            rd.wait()
            acc = (
                yrecv_ref[p, s].astype(F32)
                + xrecv_ref[p, s + 1].astype(F32)
                + partv
            )
            if s < 2:
                ysend_ref[p] = acc.astype(BF16)
            else:
                qbuf_ref[p] = acc

        def z_init(p):
            zsend_ref[p] = qbuf_ref[p, pl.ds(z * ZC, ZC), :].astype(BF16)

        def z_step_start(p, s):
            rd = rdma(
                zsend_ref.at[p], zrecv_ref.at[p, s], p, ZRS[s], z_next
            )
            rd.start()
            return rd

        def z_step_finish(p, s, rd):
            rd.wait()
            c = lax.rem(z + (3 - s), 4)
            acc = (
                zrecv_ref[p, s].astype(F32)
                + qbuf_ref[p, pl.ds(c * ZC, ZC), :]
            )
            if s < 2:
                zsend_ref[p] = acc.astype(BF16)
            else:
                fin = jnp.maximum(acc, 0.0)
                g0 = myrow0 + yc_own * QTR + zc_own * ZC
                out_ref[pl.ds(g0, ZC), pl.ds(p * HN, HN)] = fin
                zagbuf_ref[p, 0] = fin.astype(BF16)

        def zag_start(p, t):
            rd = rdma(
                zagbuf_ref.at[p, t], zagbuf_ref.at[p, t + 1],
                p, ZAG[t], z_next,
            )
            rd.start()
            return rd

        def zag_store(p, t):
            zc_r = lax.rem(z + (4 - t), 4)
            out_ref[
                pl.ds(myrow0 + yc_own * QTR + zc_r * ZC, ZC),
                pl.ds(p * HN, HN),
            ] = zagbuf_ref[p, t + 1].astype(F32)

        def assemble_quarter(p):
            for k in range(4):
                zc = lax.rem(z + (1 - k) + 4, 4)
                yagbuf_ref[p, 0, pl.ds(zc * ZC, ZC), :] = zagbuf_ref[p, k]

        def yag_start(p, t):
            rd = rdma(
                yagbuf_ref.at[p, t], yagbuf_ref.at[p, t + 1],
                p, YAG[t], y_next,
            )
            rd.start()
            return rd

        def yag_store(p, t):
            yc_r = lax.rem(y + (4 - t), 4)
            out_ref[
                pl.ds(myrow0 + yc_r * QTR, QTR), pl.ds(p * HN, HN)
            ] = yagbuf_ref[p, t + 1].astype(F32)

        def xag_start(p, t):
            rd = rdma(
                yagbuf_ref.at[p, t], xagrecv_ref.at[p, t],
                p, XAG[t], x_partner,
            )
            rd.start()
            return rd

        def xag_store(p, t):
            if t == 0:
                pc = yc_own
            else:
                pc = lax.rem(y + (5 - t), 4)
            out_ref[
                pl.ds(prow0 + pc * QTR, QTR), pl.ds(p * HN, HN)
            ] = xagrecv_ref[p, t].astype(F32)

        xr0 = xrs_start(0)
        xr1 = xrs_start(1)

        y_init(0, xr0)
        for s in range(3):
            rd = y_step_start(0, s)
            y_step_finish(0, s, rd, xr0)

        y_init(1, xr1)
        z_init(0)
        for k in range(3):
            yrd = y_step_start(1, k)
            zrd = z_step_start(0, k)
            z_step_finish(0, k, zrd)
            y_step_finish(1, k, yrd, xr1)

        store_q = []

        def flush():
            for fn in store_q:
                fn()
            store_q.clear()

        z_init(1)
        for k in range(3):
            zagrd = zag_start(0, k)
            zrd = z_step_start(1, k)
            flush()
            zagrd.wait()
            store_q.append(functools.partial(zag_store, 0, k))
            z_step_finish(1, k, zrd)
        assemble_quarter(0)

        xag_pending0 = xag_start(0, 0)
        for k in range(3):
            yagrd = yag_start(0, k)
            zagrd = zag_start(1, k)
            flush()
            yagrd.wait()
            nxt = xag_start(0, k + 1)
            store_q.append(functools.partial(yag_store, 0, k))
            xag_pending0.wait()
            store_q.append(functools.partial(xag_store, 0, k))
            xag_pending0 = nxt
            zagrd.wait()
            store_q.append(functools.partial(zag_store, 1, k))
        assemble_quarter(1)

        xag_pending1 = xag_start(1, 0)
        for k in range(3):
            yagrd = yag_start(1, k)
            flush()
            if k == 0:
                xag_pending0.wait()
                store_q.append(functools.partial(xag_store, 0, 3))
            yagrd.wait()
            nxt = xag_start(1, k + 1)
            store_q.append(functools.partial(yag_store, 1, k))
            xag_pending1.wait()
            store_q.append(functools.partial(xag_store, 1, k))
            xag_pending1 = nxt
        flush()
        xag_pending1.wait()
        xag_store(1, 3)

        @functools.partial(
            pl.run_scoped, second_barrier=pltpu.SemaphoreType.REGULAR
        )
        def _(second_barrier):
            for nbr in (x_partner, y_next, y_prev, z_next, z_prev):
                pl.semaphore_signal(
                    second_barrier, inc=1,
                    device_id=(nbr,), device_id_type=pl.DeviceIdType.MESH,
                )
            pl.semaphore_wait(second_barrier, 5)

    return pl.pallas_call(
        body,
        out_shape=jax.ShapeDtypeStruct((M, N), F32),
        in_specs=[
            pl.BlockSpec(memory_space=pltpu.VMEM),
            pl.BlockSpec(memory_space=pltpu.VMEM),
        ],
        out_specs=pl.BlockSpec(memory_space=pltpu.VMEM),
        scratch_shapes=[
            pltpu.VMEM((2, 4, QTR, HN), BF16),
            pltpu.VMEM((2, 4, QTR, HN), BF16),
            pltpu.VMEM((2, QTR, HN), BF16),
            pltpu.VMEM((2, 3, QTR, HN), BF16),
            pltpu.VMEM((2, QTR, HN), F32),
            pltpu.VMEM((2, ZC, HN), BF16),
            pltpu.VMEM((2, 3, ZC, HN), BF16),
            pltpu.VMEM((2, 4, ZC, HN), BF16),
            pltpu.VMEM((2, 4, QTR, HN), BF16),
            pltpu.VMEM((2, 4, QTR, HN), BF16),
            pltpu.SemaphoreType.DMA((40,)),
            pltpu.SemaphoreType.DMA((40,)),
        ],
        compiler_params=pltpu.CompilerParams(
            collective_id=0, vmem_limit_bytes=100 * 1024 * 1024
        ),
    )(A, B)


# device time: 152638 ns/iter; 1.1091x vs baseline; 1.0482x over previous
import functools

import jax
import jax.numpy as jnp
from jax import lax
from jax.experimental import pallas as pl
from jax.experimental.pallas import tpu as pltpu

M = 2048
N = 2048
HN = N // 2
HALF = M // 2
QTR = HALF // 4
ZC = QTR // 4

F32 = jnp.float32
BF16 = jnp.bfloat16

XRS = (0, 1, 2, 3)
YRS = (4, 5, 6)
ZRS = (7, 8, 9)
ZAG = (10, 11, 12)
YAG = (13, 14, 15)
XAG = (16, 17, 18, 19)


def kernel(A, B):
    def body(
        a_ref,
        b_ref,
        out_ref,
        xsend_ref,
        xrecv_ref,
        ysend_ref,
        yrecv_ref,
        qbuf_ref,
        zsend_ref,
        zrecv_ref,
        zagbuf_ref,
        yagbuf_ref,
        xagrecv_ref,
        send_sems,
        recv_sems,
    ):
        me = lax.axis_index("i")
        z = me // 8
        q = lax.rem(me, 8)
        y = q // 2
        r4 = lax.rem(q, 4)
        x = jnp.where((r4 == 1) | (r4 == 2), 1, 0)

        def q_of(x_, y_):
            return 2 * y_ + jnp.where(lax.rem(y_, 2) == 0, x_, 1 - x_)

        x_partner = z * 8 + (q + 1 - 2 * lax.rem(q, 2))
        y_next = z * 8 + q_of(x, lax.rem(y + 1, 4))
        y_prev = z * 8 + q_of(x, lax.rem(y + 3, 4))
        z_next = lax.rem(z + 1, 4) * 8 + q
        z_prev = lax.rem(z + 3, 4) * 8 + q

        myrow0 = x * HALF
        prow0 = (1 - x) * HALF
        yc_own = lax.rem(y + 1, 4)
        zc_own = lax.rem(z + 1, 4)

        barrier_sem = pltpu.get_barrier_semaphore()
        for nbr in (x_partner, y_next, y_prev, z_next, z_prev):
            pl.semaphore_signal(
                barrier_sem, inc=1,
                device_id=(nbr,), device_id_type=pl.DeviceIdType.MESH,
            )
        pl.semaphore_wait(barrier_sem, 5)

        def rdma(src, dst, p, idx, target):
            return pltpu.make_async_remote_copy(
                src_ref=src,
                dst_ref=dst,
                send_sem=send_sems.at[p * 20 + idx],
                recv_sem=recv_sems.at[p * 20 + idx],
                device_id=(target,),
                device_id_type=pl.DeviceIdType.MESH,
            )

        def part_rows(p, row0, nrows):
            return jnp.dot(
                a_ref[pl.ds(row0, nrows), :],
                b_ref[:, pl.ds(p * HN, HN)],
                preferred_element_type=F32,
            )

        def xrs_start(p):
            rds = []
            for j in range(4):
                cj = lax.rem(y + (4 - j), 4)
                xsend_ref[p, j] = (
                    part_rows(p, prow0 + cj * QTR, QTR).astype(BF16)
                )
                rd = rdma(
                    xsend_ref.at[p, j], xrecv_ref.at[p, j],
                    p, XRS[j], x_partner,
                )
                rd.start()
                rds.append(rd)
            return rds

        def y_init(p, xr):
            xr[0].wait()
            part0 = part_rows(p, myrow0 + y * QTR, QTR)
            ysend_ref[p] = (xrecv_ref[p, 0].astype(F32) + part0).astype(BF16)

        def y_step_start(p, s):
            rd = rdma(
                ysend_ref.at[p], yrecv_ref.at[p, s], p, YRS[s], y_next
            )
            rd.start()
            return rd

        def y_step_finish(p, s, rd, xr):
            c = lax.rem(y + (3 - s), 4)
            partv = part_rows(p, myrow0 + c * QTR, QTR)
            xr[s + 1].wait()
            rd.wait()
            acc = (
                yrecv_ref[p, s].astype(F32)
                + xrecv_ref[p, s + 1].astype(F32)
                + partv
            )
            if s < 2:
                ysend_ref[p] = acc.astype(BF16)
            else:
                qbuf_ref[p] = acc

        def z_init(p):
            zsend_ref[p] = qbuf_ref[p, pl.ds(z * ZC, ZC), :].astype(BF16)

        def z_step_start(p, s):
            rd = rdma(
                zsend_ref.at[p], zrecv_ref.at[p, s], p, ZRS[s], z_next
            )
            rd.start()
            return rd

        def z_step_finish(p, s, rd):
            rd.wait()
            c = lax.rem(z + (3 - s), 4)
            acc = (
                zrecv_ref[p, s].astype(F32)
                + qbuf_ref[p, pl.ds(c * ZC, ZC), :]
            )
            if s < 2:
                zsend_ref[p] = acc.astype(BF16)
            else:
                fin = jnp.maximum(acc, 0.0)
                g0 = myrow0 + yc_own * QTR + zc_own * ZC
                out_ref[pl.ds(g0, ZC), pl.ds(p * HN, HN)] = fin
                zagbuf_ref[p, 0] = fin.astype(BF16)

        def zag_start(p, t):
            rd = rdma(
                zagbuf_ref.at[p, t], zagbuf_ref.at[p, t + 1],
                p, ZAG[t], z_next,
            )
            rd.start()
            return rd

        def zag_store(p, t):
            zc_r = lax.rem(z + (4 - t), 4)
            out_ref[
                pl.ds(myrow0 + yc_own * QTR + zc_r * ZC, ZC),
                pl.ds(p * HN, HN),
            ] = zagbuf_ref[p, t + 1].astype(F32)

        def assemble_quarter(p):
            for k in range(4):
                zc = lax.rem(z + (1 - k) + 4, 4)
                yagbuf_ref[p, 0, pl.ds(zc * ZC, ZC), :] = zagbuf_ref[p, k]

        def yag_start(p, t):
            rd = rdma(
                yagbuf_ref.at[p, t], yagbuf_ref.at[p, t + 1],
                p, YAG[t], y_next,
            )
            rd.start()
            return rd

        def yag_store(p, t):
            yc_r = lax.rem(y + (4 - t), 4)
            out_ref[
                pl.ds(myrow0 + yc_r * QTR, QTR), pl.ds(p * HN, HN)
            ] = yagbuf_ref[p, t + 1].astype(F32)

        def xag_start(p, t):
            rd = rdma(
                yagbuf_ref.at[p, t], xagrecv_ref.at[p, t],
                p, XAG[t], x_partner,
            )
            rd.start()
            return rd

        def xag_store(p, t):
            if t == 0:
                pc = yc_own
            else:
                pc = lax.rem(y + (5 - t), 4)
            out_ref[
                pl.ds(prow0 + pc * QTR, QTR), pl.ds(p * HN, HN)
            ] = xagrecv_ref[p, t].astype(F32)

        xr0 = xrs_start(0)
        xr1 = xrs_start(1)

        y_init(0, xr0)
        for s in range(3):
            rd = y_step_start(0, s)
            y_step_finish(0, s, rd, xr0)

        y_init(1, xr1)
        z_init(0)
        for k in range(3):
            yrd = y_step_start(1, k)
            zrd = z_step_start(0, k)
            z_step_finish(0, k, zrd)
            y_step_finish(1, k, yrd, xr1)

        store_q = []

        def flush():
            for fn in store_q:
                fn()
            store_q.clear()

        z_init(1)
        for k in range(3):
            zagrd = zag_start(0, k)
            zrd = z_step_start(1, k)
            flush()
            zagrd.wait()
            store_q.append(functools.partial(zag_store, 0, k))
            z_step_finish(1, k, zrd)
        assemble_quarter(0)

        xag0 = [xag_start(0, 0)]
        yag0_rd = yag_start(0, 0)
        zag1_rd = zag_start(1, 0)

        for k in range(2):
            flush()
            zag1_rd.wait()
            store_q.append(functools.partial(zag_store, 1, k))
            zag1_rd = zag_start(1, k + 1)
            yag0_rd.wait()
            store_q.append(functools.partial(yag_store, 0, k))
            xag0.append(xag_start(0, k + 1))
            yag0_rd = yag_start(0, k + 1)
            xag0[k].wait()
            store_q.append(functools.partial(xag_store, 0, k))

        flush()
        zag1_rd.wait()
        store_q.append(functools.partial(zag_store, 1, 2))
        assemble_quarter(1)
        yag1_rd = yag_start(1, 0)
        xag1 = [xag_start(1, 0)]
        yag0_rd.wait()
        store_q.append(functools.partial(yag_store, 0, 2))
        xag0.append(xag_start(0, 3))
        xag0[2].wait()
        store_q.append(functools.partial(xag_store, 0, 2))

        for k in range(2):
            flush()
            yag1_rd.wait()
            store_q.append(functools.partial(yag_store, 1, k))
            xag1.append(xag_start(1, k + 1))
            yag1_rd = yag_start(1, k + 1)
            if k == 0:
                xag0[3].wait()
                store_q.append(functools.partial(xag_store, 0, 3))
            xag1[k].wait()
            store_q.append(functools.partial(xag_store, 1, k))

        flush()
        yag1_rd.wait()
        yag_store(1, 2)
        xag1.append(xag_start(1, 3))
        xag1[2].wait()
        xag_store(1, 2)
        xag1[3].wait()
        xag_store(1, 3)

        @functools.partial(
            pl.run_scoped, second_barrier=pltpu.SemaphoreType.REGULAR
        )
        def _(second_barrier):
            for nbr in (x_partner, y_next, y_prev, z_next, z_prev):
                pl.semaphore_signal(
                    second_barrier, inc=1,
                    device_id=(nbr,), device_id_type=pl.DeviceIdType.MESH,
                )
            pl.semaphore_wait(second_barrier, 5)

    return pl.pallas_call(
        body,
        out_shape=jax.ShapeDtypeStruct((M, N), F32),
        in_specs=[
            pl.BlockSpec(memory_space=pltpu.VMEM),
            pl.BlockSpec(memory_space=pltpu.VMEM),
        ],
        out_specs=pl.BlockSpec(memory_space=pltpu.VMEM),
        scratch_shapes=[
            pltpu.VMEM((2, 4, QTR, HN), BF16),
            pltpu.VMEM((2, 4, QTR, HN), BF16),
            pltpu.VMEM((2, QTR, HN), BF16),
            pltpu.VMEM((2, 3, QTR, HN), BF16),
            pltpu.VMEM((2, QTR, HN), F32),
            pltpu.VMEM((2, ZC, HN), BF16),
            pltpu.VMEM((2, 3, ZC, HN), BF16),
            pltpu.VMEM((2, 4, ZC, HN), BF16),
            pltpu.VMEM((2, 4, QTR, HN), BF16),
            pltpu.VMEM((2, 4, QTR, HN), BF16),
            pltpu.SemaphoreType.DMA((40,)),
            pltpu.SemaphoreType.DMA((40,)),
        ],
        compiler_params=pltpu.CompilerParams(
            collective_id=0, vmem_limit_bytes=100 * 1024 * 1024
        ),
    )(A, B)


# device time: 147303 ns/iter; 1.1493x vs baseline; 1.0362x over previous
import functools

import jax
import jax.numpy as jnp
from jax import lax
from jax.experimental import pallas as pl
from jax.experimental.pallas import tpu as pltpu

M = 2048
N = 2048
HN = N // 2
HALF = M // 2
QTR = HALF // 4
ZC = QTR // 4

F32 = jnp.float32
BF16 = jnp.bfloat16

XRS = (0, 1, 2, 3)
YRS = (4, 5, 6)
ZRS = (7, 8, 9)
ZAG = (10, 11, 12)
YAG = (13, 14, 15)
XAG = (16, 17, 18, 19)


def kernel(A, B):
    def body(
        a_ref,
        b_ref,
        out_ref,
        xsend_ref,
        xrecv_ref,
        ysend_ref,
        yrecv_ref,
        qbuf_ref,
        zsend_ref,
        zrecv_ref,
        zagbuf_ref,
        yagbuf_ref,
        xagrecv_ref,
        send_sems,
        recv_sems,
    ):
        me = lax.axis_index("i")
        z = me // 8
        q = lax.rem(me, 8)
        y = q // 2
        r4 = lax.rem(q, 4)
        x = jnp.where((r4 == 1) | (r4 == 2), 1, 0)

        def q_of(x_, y_):
            return 2 * y_ + jnp.where(lax.rem(y_, 2) == 0, x_, 1 - x_)

        x_partner = z * 8 + (q + 1 - 2 * lax.rem(q, 2))
        y_next = z * 8 + q_of(x, lax.rem(y + 1, 4))
        y_prev = z * 8 + q_of(x, lax.rem(y + 3, 4))
        z_next = lax.rem(z + 1, 4) * 8 + q
        z_prev = lax.rem(z + 3, 4) * 8 + q

        myrow0 = x * HALF
        prow0 = (1 - x) * HALF
        yc_own = lax.rem(y + 1, 4)
        zc_own = lax.rem(z + 1, 4)

        barrier_sem = pltpu.get_barrier_semaphore()
        for nbr in (x_partner, y_next, y_prev, z_next, z_prev):
            pl.semaphore_signal(
                barrier_sem, inc=1,
                device_id=(nbr,), device_id_type=pl.DeviceIdType.MESH,
            )
        pl.semaphore_wait(barrier_sem, 5)

        def rdma(src, dst, p, idx, target):
            return pltpu.make_async_remote_copy(
                src_ref=src,
                dst_ref=dst,
                send_sem=send_sems.at[p * 20 + idx],
                recv_sem=recv_sems.at[p * 20 + idx],
                device_id=(target,),
                device_id_type=pl.DeviceIdType.MESH,
            )

        def part_rows(p, row0, nrows):
            return jnp.dot(
                a_ref[pl.ds(row0, nrows), :],
                b_ref[:, pl.ds(p * HN, HN)],
                preferred_element_type=F32,
            )

        def xrs_start(p):
            rds = []
            for j in range(4):
                cj = lax.rem(y + (4 - j), 4)
                xsend_ref[p, j] = (
                    part_rows(p, prow0 + cj * QTR, QTR).astype(BF16)
                )
                rd = rdma(
                    xsend_ref.at[p, j], xrecv_ref.at[p, j],
                    p, XRS[j], x_partner,
                )
                rd.start()
                rds.append(rd)
            return rds

        def y_init(p, xr):
            xr[0].wait()
            part0 = part_rows(p, myrow0 + y * QTR, QTR)
            ysend_ref[p] = (xrecv_ref[p, 0].astype(F32) + part0).astype(BF16)

        def y_step_start(p, s):
            rd = rdma(
                ysend_ref.at[p], yrecv_ref.at[p, s], p, YRS[s], y_next
            )
            rd.start()
            return rd

        def y_step_finish(p, s, rd, xr):
            c = lax.rem(y + (3 - s), 4)
            partv = part_rows(p, myrow0 + c * QTR, QTR)
            xr[s + 1].wait()
            rd.wait()
            acc = (
                yrecv_ref[p, s].astype(F32)
                + xrecv_ref[p, s + 1].astype(F32)
                + partv
            )
            if s < 2:
                ysend_ref[p] = acc.astype(BF16)
            else:
                qbuf_ref[p] = acc

        def z_init(p):
            zsend_ref[p] = qbuf_ref[p, pl.ds(z * ZC, ZC), :].astype(BF16)

        def z_step_start(p, s):
            rd = rdma(
                zsend_ref.at[p], zrecv_ref.at[p, s], p, ZRS[s], z_next
            )
            rd.start()
            return rd

        def z_step_finish(p, s, rd):
            rd.wait()
            c = lax.rem(z + (3 - s), 4)
            acc = (
                zrecv_ref[p, s].astype(F32)
                + qbuf_ref[p, pl.ds(c * ZC, ZC), :]
            )
            if s < 2:
                zsend_ref[p] = acc.astype(BF16)
            else:
                fin = jnp.maximum(acc, 0.0)
                g0 = myrow0 + yc_own * QTR + zc_own * ZC
                out_ref[pl.ds(g0, ZC), pl.ds(p * HN, HN)] = fin
                zagbuf_ref[p, 0] = fin.astype(BF16)

        def zag_start(p, t):
            rd = rdma(
                zagbuf_ref.at[p, t], zagbuf_ref.at[p, t + 1],
                p, ZAG[t], z_next,
            )
            rd.start()
            return rd

        def zag_store(p, t):
            zc_r = lax.rem(z + (4 - t), 4)
            out_ref[
                pl.ds(myrow0 + yc_own * QTR + zc_r * ZC, ZC),
                pl.ds(p * HN, HN),
            ] = zagbuf_ref[p, t + 1].astype(F32)

        def assemble_quarter(p):
            for k in range(4):
                zc = lax.rem(z + (1 - k) + 4, 4)
                yagbuf_ref[p, 0, pl.ds(zc * ZC, ZC), :] = zagbuf_ref[p, k]

        def yag_start(p, t):
            rd = rdma(
                yagbuf_ref.at[p, t], yagbuf_ref.at[p, t + 1],
                p, YAG[t], y_next,
            )
            rd.start()
            return rd

        def yag_store(p, t):
            yc_r = lax.rem(y + (4 - t), 4)
            out_ref[
                pl.ds(myrow0 + yc_r * QTR, QTR), pl.ds(p * HN, HN)
            ] = yagbuf_ref[p, t + 1].astype(F32)

        def xag_start(p, t):
            rd = rdma(
                yagbuf_ref.at[p, t], xagrecv_ref.at[p, t],
                p, XAG[t], x_partner,
            )
            rd.start()
            return rd

        def xag_store(p, t):
            if t == 0:
                pc = yc_own
            else:
                pc = lax.rem(y + (5 - t), 4)
            out_ref[
                pl.ds(prow0 + pc * QTR, QTR), pl.ds(p * HN, HN)
            ] = xagrecv_ref[p, t].astype(F32)

        xr0 = xrs_start(0)
        xr1 = xrs_start(1)

        y_init(0, xr0)
        for s in range(3):
            rd = y_step_start(0, s)
            y_step_finish(0, s, rd, xr0)

        store_q = []

        def flush():
            for fn in store_q:
                fn()
            store_q.clear()

        y_init(1, xr1)
        z_init(0)
        zr0 = z_step_start(0, 0)
        for k in range(3):
            yrd = y_step_start(1, k)
            z_step_finish(0, k, zr0)
            if k < 2:
                zr0 = z_step_start(0, k + 1)
            else:
                za0 = zag_start(0, 0)
            y_step_finish(1, k, yrd, xr1)

        z_init(1)
        zr1 = z_step_start(1, 0)
        za0.wait()
        store_q.append(functools.partial(zag_store, 0, 0))
        za0 = zag_start(0, 1)
        z_step_finish(1, 0, zr1)
        zr1 = z_step_start(1, 1)
        za0.wait()
        store_q.append(functools.partial(zag_store, 0, 1))
        za0 = zag_start(0, 2)
        z_step_finish(1, 1, zr1)
        zr1 = z_step_start(1, 2)
        za0.wait()
        store_q.append(functools.partial(zag_store, 0, 2))
        assemble_quarter(0)

        xag0 = [xag_start(0, 0)]
        yag0_rd = yag_start(0, 0)
        z_step_finish(1, 2, zr1)
        zag1_rd = zag_start(1, 0)

        for k in range(2):
            flush()
            zag1_rd.wait()
            store_q.append(functools.partial(zag_store, 1, k))
            zag1_rd = zag_start(1, k + 1)
            yag0_rd.wait()
            store_q.append(functools.partial(yag_store, 0, k))
            xag0.append(xag_start(0, k + 1))
            yag0_rd = yag_start(0, k + 1)
            xag0[k].wait()
            store_q.append(functools.partial(xag_store, 0, k))

        flush()
        zag1_rd.wait()
        store_q.append(functools.partial(zag_store, 1, 2))
        assemble_quarter(1)
        yag1_rd = yag_start(1, 0)
        xag1 = [xag_start(1, 0)]
        yag0_rd.wait()
        store_q.append(functools.partial(yag_store, 0, 2))
        xag0.append(xag_start(0, 3))
        xag0[2].wait()
        store_q.append(functools.partial(xag_store, 0, 2))

        for k in range(2):
            flush()
            yag1_rd.wait()
            store_q.append(functools.partial(yag_store, 1, k))
            xag1.append(xag_start(1, k + 1))
            yag1_rd = yag_start(1, k + 1)
            if k == 0:
                xag0[3].wait()
                store_q.append(functools.partial(xag_store, 0, 3))
            xag1[k].wait()
            store_q.append(functools.partial(xag_store, 1, k))

        flush()
        yag1_rd.wait()
        yag_store(1, 2)
        xag1.append(xag_start(1, 3))
        xag1[2].wait()
        xag_store(1, 2)
        xag1[3].wait()
        xag_store(1, 3)

        @functools.partial(
            pl.run_scoped, second_barrier=pltpu.SemaphoreType.REGULAR
        )
        def _(second_barrier):
            for nbr in (x_partner, y_next, y_prev, z_next, z_prev):
                pl.semaphore_signal(
                    second_barrier, inc=1,
                    device_id=(nbr,), device_id_type=pl.DeviceIdType.MESH,
                )
            pl.semaphore_wait(second_barrier, 5)

    return pl.pallas_call(
        body,
        out_shape=jax.ShapeDtypeStruct((M, N), F32),
        in_specs=[
            pl.BlockSpec(memory_space=pltpu.VMEM),
            pl.BlockSpec(memory_space=pltpu.VMEM),
        ],
        out_specs=pl.BlockSpec(memory_space=pltpu.VMEM),
        scratch_shapes=[
            pltpu.VMEM((2, 4, QTR, HN), BF16),
            pltpu.VMEM((2, 4, QTR, HN), BF16),
            pltpu.VMEM((2, QTR, HN), BF16),
            pltpu.VMEM((2, 3, QTR, HN), BF16),
            pltpu.VMEM((2, QTR, HN), F32),
            pltpu.VMEM((2, ZC, HN), BF16),
            pltpu.VMEM((2, 3, ZC, HN), BF16),
            pltpu.VMEM((2, 4, ZC, HN), BF16),
            pltpu.VMEM((2, 4, QTR, HN), BF16),
            pltpu.VMEM((2, 4, QTR, HN), BF16),
            pltpu.SemaphoreType.DMA((40,)),
            pltpu.SemaphoreType.DMA((40,)),
        ],
        compiler_params=pltpu.CompilerParams(
            collective_id=0, vmem_limit_bytes=100 * 1024 * 1024
        ),
    )(A, B)
